# Initial kernel scaffold; baseline (speedup 1.0000x reference)
import jax
import jax.numpy as jnp
from jax.experimental import pallas as pl


def kernel(source, target, proj):
    raise NotImplementedError("write your pallas kernel here")


# speedup vs baseline: 1.4980x; 1.4980x over previous
import functools

import jax
import jax.numpy as jnp
from jax.experimental import pallas as pl

N, D, NPROJ = 16384, 512, 1000
PPAD = 1024
PBLK = 128
RBLK = 2048


def _project_kernel(s_ref, t_ref, p_ref, xp_ref, yp_ref):
    p = p_ref[...]
    nrm2 = jnp.sum(p * p, axis=0, keepdims=True)
    inv = jax.lax.rsqrt(jnp.where(nrm2 > 0, nrm2, 1.0))
    pn = p * inv
    xp_ref[...] = jnp.dot(s_ref[...], pn, preferred_element_type=jnp.float32)
    yp_ref[...] = jnp.dot(t_ref[...], pn, preferred_element_type=jnp.float32)


@functools.partial(jax.jit, static_argnums=())
def _project(source, target, proj_padded):
    grid = (PPAD // PBLK, N // RBLK)
    return pl.pallas_call(
        _project_kernel,
        grid=grid,
        in_specs=[
            pl.BlockSpec((RBLK, D), lambda i, j: (j, 0)),
            pl.BlockSpec((RBLK, D), lambda i, j: (j, 0)),
            pl.BlockSpec((D, PBLK), lambda i, j: (0, i)),
        ],
        out_specs=[
            pl.BlockSpec((RBLK, PBLK), lambda i, j: (j, i)),
            pl.BlockSpec((RBLK, PBLK), lambda i, j: (j, i)),
        ],
        out_shape=[
            jax.ShapeDtypeStruct((N, PPAD), jnp.float32),
            jax.ShapeDtypeStruct((N, PPAD), jnp.float32),
        ],
    )(source, target, proj_padded)


def kernel(source, target, proj):
    proj_padded = jnp.pad(proj, ((0, 0), (0, PPAD - NPROJ)))
    xp, yp = _project(source, target, proj_padded)
    xs = jnp.sort(xp, axis=0)
    ys = jnp.sort(yp, axis=0)
    total = jnp.sum((xs - ys) ** 2)
    return jnp.sqrt(total / (N * NPROJ))


# speedup vs baseline: 6.1840x; 4.1281x over previous
import functools

import jax
import jax.numpy as jnp
import numpy as np
from jax.experimental import pallas as pl
from jax.experimental.pallas import tpu as pltpu

N, D, NPROJ = 16384, 512, 1000
PPAD = 1024
PBLK = 128
RBLK = 2048
TILE = 512


def _project_kernel(s_ref, t_ref, p_ref, xp_ref, yp_ref):
    p = p_ref[...]
    nrm2 = jnp.sum(p * p, axis=0, keepdims=True)
    inv = jax.lax.rsqrt(jnp.where(nrm2 > 0, nrm2, 1.0))
    pn = p * inv
    xp_ref[...] = jnp.dot(s_ref[...], pn, preferred_element_type=jnp.float32)
    yp_ref[...] = jnp.dot(t_ref[...], pn, preferred_element_type=jnp.float32)


def _project(source, target, proj_padded):
    grid = (PPAD // PBLK, N // RBLK)
    return pl.pallas_call(
        _project_kernel,
        grid=grid,
        in_specs=[
            pl.BlockSpec((RBLK, D), lambda i, j: (j, 0)),
            pl.BlockSpec((RBLK, D), lambda i, j: (j, 0)),
            pl.BlockSpec((D, PBLK), lambda i, j: (0, i)),
        ],
        out_specs=[
            pl.BlockSpec((RBLK, PBLK), lambda i, j: (j, i)),
            pl.BlockSpec((RBLK, PBLK), lambda i, j: (j, i)),
        ],
        out_shape=[
            jax.ShapeDtypeStruct((N, PPAD), jnp.float32),
            jax.ShapeDtypeStruct((N, PPAD), jnp.float32),
        ],
    )(source, target, proj_padded)


def _ce_stage(x, j, k, asc_scalar=None):
    r = x.shape[0]
    i = jax.lax.broadcasted_iota(jnp.int32, (r, 1), 0)
    bitj = (i & j) != 0
    up = jnp.concatenate([x[j:], x[:j]], axis=0)
    down = jnp.concatenate([x[-j:], x[:-j]], axis=0)
    partner = jnp.where(bitj, down, up)
    mn = jnp.minimum(x, partner)
    mx = jnp.maximum(x, partner)
    if asc_scalar is None:
        take_min = ((i & k) == 0) != bitj
    else:
        take_min = jnp.logical_xor(asc_scalar, bitj)
    return jnp.where(take_min, mn, mx)


def _local_bitonic_sort(x, final_asc):
    r = x.shape[0]
    k = 2
    while k <= r:
        j = k // 2
        while j >= 1:
            x = _ce_stage(x, j, k, asc_scalar=final_asc if k == r else None)
            j //= 2
        k *= 2
    return x


def _local_merge(x, asc_scalar):
    j = x.shape[0] // 2
    while j >= 1:
        x = _ce_stage(x, j, 0, asc_scalar=asc_scalar)
        j //= 2
    return x


def _make_sort_kernel(n, tile, pblk):
    tiles = n // tile
    chunks = (n // 2) // tile

    def _sort_kernel(xp_ref, yp_ref, out_ref, s_ref):
        def _al(i):
            return pl.multiple_of(i, tile)

        def phase_a(m, carry):
            base = _al(m * tile)
            x = jnp.concatenate(
                [xp_ref[pl.ds(base, tile), :], yp_ref[pl.ds(base, tile), :]],
                axis=1)
            x = _local_bitonic_sort(x, final_asc=(m & 1) == 0)
            s_ref[pl.ds(base, tile), :] = x
            return carry
        jax.lax.fori_loop(0, tiles, phase_a, 0)

        k = 2 * tile
        while k <= n:
            j = k // 2
            while j >= tile:
                lj = j.bit_length() - 1

                def phase_b_global(c, carry, lj=lj, j=j, k=k):
                    cb = c * tile
                    q = cb >> lj
                    rr = cb & (j - 1)
                    ia = _al((q << (lj + 1)) + rr)
                    ib = _al(ia + j)
                    a = s_ref[pl.ds(ia, tile), :]
                    b = s_ref[pl.ds(ib, tile), :]
                    asc = (ia & k) == 0
                    mn = jnp.minimum(a, b)
                    mx = jnp.maximum(a, b)
                    s_ref[pl.ds(ia, tile), :] = jnp.where(asc, mn, mx)
                    s_ref[pl.ds(ib, tile), :] = jnp.where(asc, mx, mn)
                    return carry
                jax.lax.fori_loop(0, chunks, phase_b_global, 0)
                j //= 2

            def phase_b_local(m, carry, k=k):
                base = _al(m * tile)
                x = s_ref[pl.ds(base, tile), :]
                asc = (base & k) == 0
                x = _local_merge(x, asc)
                s_ref[pl.ds(base, tile), :] = x
                return carry
            jax.lax.fori_loop(0, tiles, phase_b_local, 0)
            k *= 2

        def reduce_tile(m, acc):
            x = s_ref[pl.ds(_al(m * tile), tile), :]
            d = x[:, :pblk] - x[:, pblk:]
            return acc + jnp.sum(d * d, axis=0, keepdims=True)
        acc = jax.lax.fori_loop(
            0, tiles, reduce_tile, jnp.zeros((1, pblk), jnp.float32))
        out_ref[...] = acc[None]

    return _sort_kernel


def _sorted_sq_diff(xp, yp, n=N, tile=TILE, pblk=PBLK):
    ppad = xp.shape[1]
    grid = (ppad // pblk,)
    return pl.pallas_call(
        _make_sort_kernel(n, tile, pblk),
        grid=grid,
        in_specs=[
            pl.BlockSpec((n, pblk), lambda i: (0, i)),
            pl.BlockSpec((n, pblk), lambda i: (0, i)),
        ],
        out_specs=pl.BlockSpec((1, 1, pblk), lambda i: (i, 0, 0)),
        out_shape=jax.ShapeDtypeStruct((ppad // pblk, 1, pblk), jnp.float32),
        scratch_shapes=[pltpu.VMEM((n, 2 * pblk), jnp.float32)],
    )(xp, yp)


def kernel(source, target, proj):
    proj_padded = jnp.pad(proj, ((0, 0), (0, PPAD - NPROJ)))
    xp, yp = _project(source, target, proj_padded)
    partial = _sorted_sq_diff(xp, yp)
    return jnp.sqrt(jnp.sum(partial) / (N * NPROJ))


# speedup vs baseline: 11.7507x; 1.9002x over previous
import jax
import jax.numpy as jnp
from jax.experimental import pallas as pl
from jax.experimental.pallas import tpu as pltpu

N, D, NPROJ = 16384, 512, 1000
PPAD = 1024
PBLK = 128
RBLK = 2048
TILE = 512


def _project_kernel(s_ref, t_ref, p_ref, xp_ref, yp_ref):
    p = p_ref[...]
    nrm2 = jnp.sum(p * p, axis=0, keepdims=True)
    inv = jax.lax.rsqrt(jnp.where(nrm2 > 0, nrm2, 1.0))
    pn = p * inv
    xp_ref[...] = jnp.dot(s_ref[...], pn, preferred_element_type=jnp.float32)
    yp_ref[...] = jnp.dot(t_ref[...], pn, preferred_element_type=jnp.float32)


def _project(source, target, proj_padded):
    grid = (PPAD // PBLK, N // RBLK)
    return pl.pallas_call(
        _project_kernel,
        grid=grid,
        in_specs=[
            pl.BlockSpec((RBLK, D), lambda i, j: (j, 0)),
            pl.BlockSpec((RBLK, D), lambda i, j: (j, 0)),
            pl.BlockSpec((D, PBLK), lambda i, j: (0, i)),
        ],
        out_specs=[
            pl.BlockSpec((RBLK, PBLK), lambda i, j: (j, i)),
            pl.BlockSpec((RBLK, PBLK), lambda i, j: (j, i)),
        ],
        out_shape=[
            jax.ShapeDtypeStruct((N, PPAD), jnp.float32),
            jax.ShapeDtypeStruct((N, PPAD), jnp.float32),
        ],
    )(source, target, proj_padded)


def _ce_small(x, j, take_min_if_bit_clear):
    r = x.shape[0]
    i = jax.lax.broadcasted_iota(jnp.int32, (r, 1), 0)
    bitj = (i & j) != 0
    up = jnp.concatenate([x[j:], x[:j]], axis=0)
    down = jnp.concatenate([x[-j:], x[:-j]], axis=0)
    partner = jnp.where(bitj, down, up)
    mn = jnp.minimum(x, partner)
    mx = jnp.maximum(x, partner)
    take_min = bitj != take_min_if_bit_clear
    return jnp.where(take_min, mn, mx)


def _sort8(x, asc):
    for k, j in ((2, 1), (4, 2), (4, 1)):
        r = x.shape[0]
        i = jax.lax.broadcasted_iota(jnp.int32, (r, 1), 0)
        bitj = (i & j) != 0
        up = jnp.concatenate([x[j:], x[:j]], axis=0)
        down = jnp.concatenate([x[-j:], x[:-j]], axis=0)
        partner = jnp.where(bitj, down, up)
        mn = jnp.minimum(x, partner)
        mx = jnp.maximum(x, partner)
        take_min = ((i & k) == 0) != bitj
        x = jnp.where(take_min, mn, mx)
    for j in (4, 2, 1):
        x = _ce_small(x, j, asc)
    return x


def _merge_val(x, asc):
    r = x.shape[0]
    if r == 8:
        for j in (4, 2, 1):
            x = _ce_small(x, j, asc)
        return x
    h = r // 2
    a, b = x[:h], x[h:]
    mn = jnp.minimum(a, b)
    mx = jnp.maximum(a, b)
    lo, hi = (mn, mx) if asc else (mx, mn)
    return jnp.concatenate([_merge_val(lo, asc), _merge_val(hi, asc)], axis=0)


def _sort_val(x, asc):
    r = x.shape[0]
    if r == 8:
        return _sort8(x, asc)
    h = r // 2
    a = _sort_val(x[:h], True)
    b = _sort_val(x[h:], False)
    return _merge_val(jnp.concatenate([a, b], axis=0), asc)


def _make_sort_kernel(n, tile, pblk):
    tiles = n // tile
    chunks = (n // 2) // tile
    lt = tile.bit_length() - 1

    def _sort_kernel(xp_ref, yp_ref, out_ref, s_ref):
        def _al(i):
            return pl.multiple_of(i, tile)

        def _load_tile(m):
            base = _al(m * tile)
            return jnp.concatenate(
                [xp_ref[pl.ds(base, tile), :], yp_ref[pl.ds(base, tile), :]],
                axis=1)

        def phase_a(p, carry):
            for par, asc in ((0, True), (1, False)):
                m = p * 2 + par
                s_ref[pl.ds(_al(m * tile), tile), :] = _sort_val(
                    _load_tile(m), asc)
            return carry
        jax.lax.fori_loop(0, tiles // 2, phase_a, 0)

        k = 2 * tile
        while k <= n:
            lk = k.bit_length() - 1
            j = k // 2
            while j >= tile:
                lj = j.bit_length() - 1
                csz = k >> (lt + 1)

                def body(c, asc, lj=lj, j=j):
                    cb = c << lt
                    q = cb >> lj
                    rr = cb & (j - 1)
                    ia = _al((q << (lj + 1)) + rr)
                    ib = _al(ia + j)
                    a = s_ref[pl.ds(ia, tile), :]
                    b = s_ref[pl.ds(ib, tile), :]
                    mn = jnp.minimum(a, b)
                    mx = jnp.maximum(a, b)
                    lo, hi = (mn, mx) if asc else (mx, mn)
                    s_ref[pl.ds(ia, tile), :] = lo
                    s_ref[pl.ds(ib, tile), :] = hi

                if csz >= chunks:
                    def phase_bg_all(c, carry, body=body):
                        body(c, True)
                        return carry
                    jax.lax.fori_loop(0, chunks, phase_bg_all, 0)
                else:
                    def phase_bg(p, carry, body=body, csz=csz):
                        c_asc = ((p // csz) * 2 + 0) * csz + p % csz
                        body(c_asc, True)
                        body(c_asc + csz, False)
                        return carry
                    jax.lax.fori_loop(0, chunks // 2, phase_bg, 0)
                j //= 2

            run = k >> lt

            def merge_tile(m, asc):
                base = _al(m * tile)
                s_ref[pl.ds(base, tile), :] = _merge_val(
                    s_ref[pl.ds(base, tile), :], asc)

            if run >= tiles:
                def phase_bl_all(m, carry):
                    merge_tile(m, True)
                    return carry
                jax.lax.fori_loop(0, tiles, phase_bl_all, 0)
            else:
                def phase_bl(p, carry, run=run):
                    m_asc = ((p // run) * 2 + 0) * run + p % run
                    merge_tile(m_asc, True)
                    merge_tile(m_asc + run, False)
                    return carry
                jax.lax.fori_loop(0, tiles // 2, phase_bl, 0)
            k *= 2

        def reduce_tile(m, acc):
            x = s_ref[pl.ds(_al(m * tile), tile), :]
            d = x[:, :pblk] - x[:, pblk:]
            return acc + jnp.sum(d * d, axis=0, keepdims=True)
        acc = jax.lax.fori_loop(
            0, tiles, reduce_tile, jnp.zeros((1, pblk), jnp.float32))
        out_ref[...] = acc[None]

    return _sort_kernel


def _sorted_sq_diff(xp, yp, n=N, tile=TILE, pblk=PBLK):
    ppad = xp.shape[1]
    grid = (ppad // pblk,)
    return pl.pallas_call(
        _make_sort_kernel(n, tile, pblk),
        grid=grid,
        in_specs=[
            pl.BlockSpec((n, pblk), lambda i: (0, i)),
            pl.BlockSpec((n, pblk), lambda i: (0, i)),
        ],
        out_specs=pl.BlockSpec((1, 1, pblk), lambda i: (i, 0, 0)),
        out_shape=jax.ShapeDtypeStruct((ppad // pblk, 1, pblk), jnp.float32),
        scratch_shapes=[pltpu.VMEM((n, 2 * pblk), jnp.float32)],
    )(xp, yp)


def kernel(source, target, proj):
    proj_padded = jnp.pad(proj, ((0, 0), (0, PPAD - NPROJ)))
    xp, yp = _project(source, target, proj_padded)
    partial = _sorted_sq_diff(xp, yp)
    return jnp.sqrt(jnp.sum(partial) / (N * NPROJ))


# speedup vs baseline: 12.8452x; 1.0931x over previous
import jax
import jax.numpy as jnp
from jax.experimental import pallas as pl
from jax.experimental.pallas import tpu as pltpu

N, D, NPROJ = 16384, 512, 1000
PPAD = 1024
PBLK = 128
RBLK = 2048
TILE = 512


def _project_kernel(s_ref, t_ref, p_ref, xp_ref, yp_ref):
    p = p_ref[...]
    nrm2 = jnp.sum(p * p, axis=0, keepdims=True)
    inv = jax.lax.rsqrt(jnp.where(nrm2 > 0, nrm2, 1.0))
    pn = p * inv
    xp_ref[...] = jnp.dot(s_ref[...], pn, preferred_element_type=jnp.float32)
    yp_ref[...] = jnp.dot(t_ref[...], pn, preferred_element_type=jnp.float32)


def _project(source, target, proj_padded):
    grid = (PPAD // PBLK, N // RBLK)
    return pl.pallas_call(
        _project_kernel,
        grid=grid,
        in_specs=[
            pl.BlockSpec((RBLK, D), lambda i, j: (j, 0)),
            pl.BlockSpec((RBLK, D), lambda i, j: (j, 0)),
            pl.BlockSpec((D, PBLK), lambda i, j: (0, i)),
        ],
        out_specs=[
            pl.BlockSpec((RBLK, PBLK), lambda i, j: (j, i)),
            pl.BlockSpec((RBLK, PBLK), lambda i, j: (j, i)),
        ],
        out_shape=[
            jax.ShapeDtypeStruct((N, PPAD), jnp.float32),
            jax.ShapeDtypeStruct((N, PPAD), jnp.float32),
        ],
    )(source, target, proj_padded)


def _ce_small(x, j, take_min_if_bit_clear):
    r = x.shape[0]
    i = jax.lax.broadcasted_iota(jnp.int32, (r, 1), 0)
    bitj = (i & j) != 0
    up = jnp.concatenate([x[j:], x[:j]], axis=0)
    down = jnp.concatenate([x[-j:], x[:-j]], axis=0)
    if take_min_if_bit_clear:
        a = jnp.minimum(x, up)
        b = jnp.maximum(x, down)
    else:
        a = jnp.maximum(x, up)
        b = jnp.minimum(x, down)
    return jnp.where(bitj, b, a)


def _sort8(x, asc):
    for k, j in ((2, 1), (4, 2), (4, 1)):
        r = x.shape[0]
        i = jax.lax.broadcasted_iota(jnp.int32, (r, 1), 0)
        bitj = (i & j) != 0
        up = jnp.concatenate([x[j:], x[:j]], axis=0)
        down = jnp.concatenate([x[-j:], x[:-j]], axis=0)
        partner = jnp.where(bitj, down, up)
        mn = jnp.minimum(x, partner)
        mx = jnp.maximum(x, partner)
        take_min = ((i & k) == 0) != bitj
        x = jnp.where(take_min, mn, mx)
    for j in (4, 2, 1):
        x = _ce_small(x, j, asc)
    return x


def _merge_val(x, asc):
    r = x.shape[0]
    if r == 8:
        for j in (4, 2, 1):
            x = _ce_small(x, j, asc)
        return x
    h = r // 2
    a, b = x[:h], x[h:]
    mn = jnp.minimum(a, b)
    mx = jnp.maximum(a, b)
    lo, hi = (mn, mx) if asc else (mx, mn)
    return jnp.concatenate([_merge_val(lo, asc), _merge_val(hi, asc)], axis=0)


def _sort_val(x, asc):
    r = x.shape[0]
    if r == 8:
        return _sort8(x, asc)
    h = r // 2
    a = _sort_val(x[:h], True)
    b = _sort_val(x[h:], False)
    return _merge_val(jnp.concatenate([a, b], axis=0), asc)


def _make_sort_kernel(n, tile, pblk):
    tiles = n // tile
    chunks = (n // 2) // tile
    lt = tile.bit_length() - 1

    def _sort_kernel(xp_ref, yp_ref, out_ref, s_ref):
        def _al(i):
            return pl.multiple_of(i, tile)

        def _load_tile(m):
            base = _al(m * tile)
            return jnp.concatenate(
                [xp_ref[pl.ds(base, tile), :], yp_ref[pl.ds(base, tile), :]],
                axis=1)

        def phase_a(p, carry):
            for par, asc in ((0, True), (1, False)):
                m = p * 2 + par
                s_ref[pl.ds(_al(m * tile), tile), :] = _sort_val(
                    _load_tile(m), asc)
            return carry
        jax.lax.fori_loop(0, tiles // 2, phase_a, 0)

        k = 2 * tile
        while k <= n:
            lk = k.bit_length() - 1
            j = k // 2
            while j > tile:
                lj = j.bit_length() - 1
                csz = k >> (lt + 1)

                def body(c, asc, lj=lj, j=j):
                    cb = c << lt
                    q = cb >> lj
                    rr = cb & (j - 1)
                    ia = _al((q << (lj + 1)) + rr)
                    ib = _al(ia + j)
                    a = s_ref[pl.ds(ia, tile), :]
                    b = s_ref[pl.ds(ib, tile), :]
                    mn = jnp.minimum(a, b)
                    mx = jnp.maximum(a, b)
                    lo, hi = (mn, mx) if asc else (mx, mn)
                    s_ref[pl.ds(ia, tile), :] = lo
                    s_ref[pl.ds(ib, tile), :] = hi

                if csz >= chunks:
                    def phase_bg_all(c, carry, body=body):
                        body(c, True)
                        return carry
                    jax.lax.fori_loop(0, chunks, phase_bg_all, 0)
                else:
                    def phase_bg(p, carry, body=body, csz=csz):
                        c_asc = ((p // csz) * 2 + 0) * csz + p % csz
                        body(c_asc, True)
                        body(c_asc + csz, False)
                        return carry
                    jax.lax.fori_loop(0, chunks // 2, phase_bg, 0)
                j //= 2

            csz = k >> (lt + 1)

            def fused(c, asc):
                ia = _al(c * (2 * tile))
                ib = _al(ia + tile)
                a = s_ref[pl.ds(ia, tile), :]
                b = s_ref[pl.ds(ib, tile), :]
                mn = jnp.minimum(a, b)
                mx = jnp.maximum(a, b)
                lo, hi = (mn, mx) if asc else (mx, mn)
                s_ref[pl.ds(ia, tile), :] = _merge_val(lo, asc)
                s_ref[pl.ds(ib, tile), :] = _merge_val(hi, asc)

            if csz >= chunks:
                def fused_all(c, carry):
                    fused(c, True)
                    return carry
                jax.lax.fori_loop(0, chunks, fused_all, 0)
            else:
                def fused_pair(p, carry, csz=csz):
                    c_asc = ((p // csz) * 2 + 0) * csz + p % csz
                    fused(c_asc, True)
                    fused(c_asc + csz, False)
                    return carry
                jax.lax.fori_loop(0, chunks // 2, fused_pair, 0)
            k *= 2

        def reduce_tile(m, acc):
            x = s_ref[pl.ds(_al(m * tile), tile), :]
            d = x[:, :pblk] - x[:, pblk:]
            return acc + jnp.sum(d * d, axis=0, keepdims=True)
        acc = jax.lax.fori_loop(
            0, tiles, reduce_tile, jnp.zeros((1, pblk), jnp.float32))
        out_ref[...] = acc[None]

    return _sort_kernel


def _sorted_sq_diff(xp, yp, n=N, tile=TILE, pblk=PBLK):
    ppad = xp.shape[1]
    grid = (ppad // pblk,)
    return pl.pallas_call(
        _make_sort_kernel(n, tile, pblk),
        grid=grid,
        in_specs=[
            pl.BlockSpec((n, pblk), lambda i: (0, i)),
            pl.BlockSpec((n, pblk), lambda i: (0, i)),
        ],
        out_specs=pl.BlockSpec((1, 1, pblk), lambda i: (i, 0, 0)),
        out_shape=jax.ShapeDtypeStruct((ppad // pblk, 1, pblk), jnp.float32),
        scratch_shapes=[pltpu.VMEM((n, 2 * pblk), jnp.float32)],
    )(xp, yp)


def kernel(source, target, proj):
    proj_padded = jnp.pad(proj, ((0, 0), (0, PPAD - NPROJ)))
    xp, yp = _project(source, target, proj_padded)
    partial = _sorted_sq_diff(xp, yp)
    return jnp.sqrt(jnp.sum(partial) / (N * NPROJ))


# speedup vs baseline: 13.5502x; 1.0549x over previous
import jax
import jax.numpy as jnp
from jax.experimental import pallas as pl
from jax.experimental.pallas import tpu as pltpu

N, D, NPROJ = 16384, 512, 1000
PPAD = 1024
PBLK = 128
TILE = 512


def _ce_small(x, j, take_min_if_bit_clear):
    r = x.shape[0]
    i = jax.lax.broadcasted_iota(jnp.int32, (r, 1), 0)
    bitj = (i & j) != 0
    up = jnp.concatenate([x[j:], x[:j]], axis=0)
    down = jnp.concatenate([x[-j:], x[:-j]], axis=0)
    if take_min_if_bit_clear:
        a = jnp.minimum(x, up)
        b = jnp.maximum(x, down)
    else:
        a = jnp.maximum(x, up)
        b = jnp.minimum(x, down)
    return jnp.where(bitj, b, a)


def _sort8(x, asc):
    for k, j in ((2, 1), (4, 2), (4, 1)):
        r = x.shape[0]
        i = jax.lax.broadcasted_iota(jnp.int32, (r, 1), 0)
        bitj = (i & j) != 0
        up = jnp.concatenate([x[j:], x[:j]], axis=0)
        down = jnp.concatenate([x[-j:], x[:-j]], axis=0)
        partner = jnp.where(bitj, down, up)
        mn = jnp.minimum(x, partner)
        mx = jnp.maximum(x, partner)
        take_min = ((i & k) == 0) != bitj
        x = jnp.where(take_min, mn, mx)
    for j in (4, 2, 1):
        x = _ce_small(x, j, asc)
    return x


def _merge_val(x, asc):
    r = x.shape[0]
    if r == 8:
        for j in (4, 2, 1):
            x = _ce_small(x, j, asc)
        return x
    h = r // 2
    a, b = x[:h], x[h:]
    mn = jnp.minimum(a, b)
    mx = jnp.maximum(a, b)
    lo, hi = (mn, mx) if asc else (mx, mn)
    return jnp.concatenate([_merge_val(lo, asc), _merge_val(hi, asc)], axis=0)


def _sort_val(x, asc):
    r = x.shape[0]
    if r == 8:
        return _sort8(x, asc)
    h = r // 2
    a = _sort_val(x[:h], True)
    b = _sort_val(x[h:], False)
    return _merge_val(jnp.concatenate([a, b], axis=0), asc)


def _make_fused_kernel(n, d, tile, pblk, nblk):
    tiles = n // tile
    chunks = (n // 2) // tile
    lt = tile.bit_length() - 1
    mmrows = n // (tiles // 2)

    nchunk = tiles // 2

    def _fused_kernel(s_ref, t_ref, p_ref, out_ref, mm_ref, ss_ref, st_ref,
                      sem_ref):
        i = pl.program_id(0)
        par = i % 2
        prev = (i + 1) % 2

        def _al(t):
            return pl.multiple_of(t, tile)

        p = p_ref[...]
        nrm2 = jnp.sum(p * p, axis=0, keepdims=True)
        inv = jax.lax.rsqrt(jnp.where(nrm2 > 0, nrm2, 1.0))
        pn = p * inv
        do_mm = i < nblk
        do_sort = i > 0

        def _start_chunk(q, slot):
            rb = _al(q * mmrows)
            pltpu.make_async_copy(
                s_ref.at[pl.ds(rb, mmrows), :], ss_ref.at[slot],
                sem_ref.at[slot, 0]).start()
            pltpu.make_async_copy(
                t_ref.at[pl.ds(rb, mmrows), :], st_ref.at[slot],
                sem_ref.at[slot, 1]).start()

        def _wait_chunk(q, slot):
            rb = _al(q * mmrows)
            pltpu.make_async_copy(
                s_ref.at[pl.ds(rb, mmrows), :], ss_ref.at[slot],
                sem_ref.at[slot, 0]).wait()
            pltpu.make_async_copy(
                t_ref.at[pl.ds(rb, mmrows), :], st_ref.at[slot],
                sem_ref.at[slot, 1]).wait()

        @pl.when(do_mm)
        def _():
            _start_chunk(0, 0)

        def phase_a(q, carry):
            slot = q % 2

            @pl.when(do_mm & (q + 1 < nchunk))
            def _():
                _start_chunk(q + 1, (q + 1) % 2)

            @pl.when(do_mm)
            def _():
                _wait_chunk(q, slot)
                rb = _al(q * mmrows)
                mm_ref[par, pl.ds(rb, mmrows), 0:pblk] = jnp.dot(
                    ss_ref[slot], pn, preferred_element_type=jnp.float32)
                mm_ref[par, pl.ds(rb, mmrows), pblk:2 * pblk] = jnp.dot(
                    st_ref[slot], pn, preferred_element_type=jnp.float32)

            @pl.when(do_sort)
            def _():
                for parity, asc in ((0, True), (1, False)):
                    m = q * 2 + parity
                    base = _al(m * tile)
                    mm_ref[prev, pl.ds(base, tile), :] = _sort_val(
                        mm_ref[prev, pl.ds(base, tile), :], asc)
            return carry
        jax.lax.fori_loop(0, nchunk, phase_a, 0)

        @pl.when(do_sort)
        def _sort_rest():
            k = 2 * tile
            while k <= n:
                j = k // 2
                while j > tile:
                    lj = j.bit_length() - 1
                    csz = k >> (lt + 1)

                    def body(c, asc, lj=lj, j=j):
                        cb = c << lt
                        q = cb >> lj
                        rr = cb & (j - 1)
                        ia = _al((q << (lj + 1)) + rr)
                        ib = _al(ia + j)
                        a = mm_ref[prev, pl.ds(ia, tile), :]
                        b = mm_ref[prev, pl.ds(ib, tile), :]
                        mn = jnp.minimum(a, b)
                        mx = jnp.maximum(a, b)
                        lo, hi = (mn, mx) if asc else (mx, mn)
                        mm_ref[prev, pl.ds(ia, tile), :] = lo
                        mm_ref[prev, pl.ds(ib, tile), :] = hi

                    if csz >= chunks:
                        def phase_bg_all(c, carry, body=body):
                            body(c, True)
                            return carry
                        jax.lax.fori_loop(0, chunks, phase_bg_all, 0)
                    else:
                        def phase_bg(pp, carry, body=body, csz=csz):
                            c_asc = ((pp // csz) * 2) * csz + pp % csz
                            body(c_asc, True)
                            body(c_asc + csz, False)
                            return carry
                        jax.lax.fori_loop(0, chunks // 2, phase_bg, 0)
                    j //= 2

                csz = k >> (lt + 1)

                def fused(c, asc):
                    ia = _al(c * (2 * tile))
                    ib = _al(ia + tile)
                    a = mm_ref[prev, pl.ds(ia, tile), :]
                    b = mm_ref[prev, pl.ds(ib, tile), :]
                    mn = jnp.minimum(a, b)
                    mx = jnp.maximum(a, b)
                    lo, hi = (mn, mx) if asc else (mx, mn)
                    mm_ref[prev, pl.ds(ia, tile), :] = _merge_val(lo, asc)
                    mm_ref[prev, pl.ds(ib, tile), :] = _merge_val(hi, asc)

                if csz >= chunks:
                    def fused_all(c, carry):
                        fused(c, True)
                        return carry
                    jax.lax.fori_loop(0, chunks, fused_all, 0)
                else:
                    def fused_pair(pp, carry, csz=csz):
                        c_asc = ((pp // csz) * 2) * csz + pp % csz
                        fused(c_asc, True)
                        fused(c_asc + csz, False)
                        return carry
                    jax.lax.fori_loop(0, chunks // 2, fused_pair, 0)
                k *= 2

            def reduce_tile(m, acc):
                x = mm_ref[prev, pl.ds(_al(m * tile), tile), :]
                dd = x[:, :pblk] - x[:, pblk:]
                return acc + jnp.sum(dd * dd, axis=0, keepdims=True)
            acc = jax.lax.fori_loop(
                0, tiles, reduce_tile, jnp.zeros((1, pblk), jnp.float32))
            out_ref[...] = acc[None]

    return _fused_kernel


def _swd_partials(source, target, proj_padded, n=N, d=D, tile=TILE,
                  pblk=PBLK):
    ppad = proj_padded.shape[1]
    nblk = ppad // pblk
    grid = (nblk + 1,)
    return pl.pallas_call(
        _make_fused_kernel(n, d, tile, pblk, nblk),
        grid=grid,
        in_specs=[
            pl.BlockSpec(memory_space=pltpu.MemorySpace.HBM),
            pl.BlockSpec(memory_space=pltpu.MemorySpace.HBM),
            pl.BlockSpec((d, pblk), lambda i: (0, jnp.minimum(i, nblk - 1))),
        ],
        out_specs=pl.BlockSpec(
            (1, 1, pblk), lambda i: (jnp.maximum(i - 1, 0), 0, 0)),
        out_shape=jax.ShapeDtypeStruct((nblk, 1, pblk), jnp.float32),
        scratch_shapes=[
            pltpu.VMEM((2, n, 2 * pblk), jnp.float32),
            pltpu.VMEM((2, 2 * tile, d), jnp.float32),
            pltpu.VMEM((2, 2 * tile, d), jnp.float32),
            pltpu.SemaphoreType.DMA((2, 2)),
        ],
    )(source, target, proj_padded)


def kernel(source, target, proj):
    proj_padded = jnp.pad(proj, ((0, 0), (0, PPAD - NPROJ)))
    partial = _swd_partials(source, target, proj_padded)
    return jnp.sqrt(jnp.sum(partial) / (N * NPROJ))
